# baseline (speedup 1.0000x reference)
import jax
import jax.numpy as jnp
from jax import lax
from jax.experimental import pallas as pl
from jax.experimental.pallas import tpu as pltpu
from jax.experimental.pallas import tpu_sc as plsc

F32 = jnp.float32
I32 = jnp.int32

N = 10000
E = 320000
D = 128
H = 256
HH = H // 2
NP = 10240
CHUNK = 128
CPT = 160
SB = 8
KB = 2
EP = 16 * CHUNK * CPT
NROWS_T = NP // 16
RB = 256
NCONV = 18

_mesh = plsc.VectorSubcoreMesh(core_axis_name="c", subcore_axis_name="s")


def _spmm_body(src_ref, dst_ref, y0_ref, y1_ref, out0_ref, out1_ref,
               stages, sidx, didx, acc, gsems, ssems):
    c = lax.axis_index("c")
    s = lax.axis_index("s")
    r0 = s * NROWS_T

    def run(y_hbm, out_hbm):
        def init_body(j, carry):
            r = r0 + j * CHUNK
            pltpu.sync_copy(y_hbm.at[pl.ds(r, CHUNK)], stages[0])
            pltpu.sync_copy(stages[0], acc.at[pl.ds(r, CHUNK)])
            return carry
        lax.fori_loop(0, NROWS_T // CHUNK, init_body, 0)
        plsc.subcore_barrier()

        base = s * CPT

        def super_body(k, carry):
            row0 = base + k * SB
            pltpu.sync_copy(src_ref.at[pl.ds(row0, SB)], sidx)
            pltpu.sync_copy(dst_ref.at[pl.ds(row0, SB)], didx)
            for j in range(SB):
                pltpu.async_copy(
                    y_hbm.at[sidx.at[j]], stages[0], gsems[0]).wait()
                pltpu.async_copy(
                    stages[0], acc.at[didx.at[j]], ssems[0], add=True).wait()
            return carry
        lax.fori_loop(0, CPT // SB, super_body, 0)
        plsc.subcore_barrier()

        def out_body(j, carry):
            r = r0 + j * CHUNK
            pltpu.sync_copy(acc.at[pl.ds(r, CHUNK)], stages[0])
            pltpu.sync_copy(stages[0], out_hbm.at[pl.ds(r, CHUNK)])
            return carry
        lax.fori_loop(0, NROWS_T // CHUNK, out_body, 0)

    @pl.when(c == 0)
    def _():
        run(y0_ref, out0_ref)

    @pl.when(c == 1)
    def _():
        run(y1_ref, out1_ref)


def _spmm_wrap(body):
    def wrapped(src_ref, dst_ref, y0_ref, y1_ref, out0_ref, out1_ref, *scr):
        stages = scr[:KB]
        sidx = scr[KB]
        didx = scr[KB + 1]
        acc = scr[KB + 2]
        gsems = scr[KB + 3:2 * KB + 3]
        ssems = scr[2 * KB + 3:3 * KB + 3]
        body(src_ref, dst_ref, y0_ref, y1_ref, out0_ref, out1_ref,
             stages, sidx, didx, acc, gsems, ssems)
    return wrapped


_spmm_call = pl.kernel(
    _spmm_wrap(_spmm_body),
    out_type=(jax.ShapeDtypeStruct((NP, HH), F32),
              jax.ShapeDtypeStruct((NP, HH), F32)),
    mesh=_mesh,
    scratch_types=(
        [pltpu.VMEM((CHUNK, HH), F32)] * KB
        + [pltpu.VMEM((SB, CHUNK), I32),
           pltpu.VMEM((SB, CHUNK), I32),
           pltpu.VMEM_SHARED((NP, HH), F32)]
        + [pltpu.SemaphoreType.DMA] * (2 * KB)
    ),
)


def _deg_body(dst_ref, ones_ref, zeros_ref, outa_ref, outb_ref,
              obuf, zbuf, didx, acc):
    c = lax.axis_index("c")
    s = lax.axis_index("s")
    r0 = s * NROWS_T
    pltpu.sync_copy(ones_ref, obuf)
    pltpu.sync_copy(zeros_ref, zbuf)

    def zbody(j, carry):
        r = r0 + j * CHUNK
        pltpu.sync_copy(zbuf, acc.at[pl.ds(r, CHUNK)])
        return carry
    lax.fori_loop(0, NROWS_T // CHUNK, zbody, 0)
    plsc.subcore_barrier()

    base = (c * 16 + s) * (CPT // 2)

    def ebody(i, carry):
        row = base + i
        pltpu.sync_copy(dst_ref.at[pl.ds(row, 1)], didx)
        pltpu.sync_copy(obuf, acc.at[didx.at[0]], add=True)
        return carry
    lax.fori_loop(0, CPT // 2, ebody, 0)
    plsc.subcore_barrier()

    def write_out(out_hbm):
        def obody(j, carry):
            r = r0 + j * CHUNK
            pltpu.sync_copy(acc.at[pl.ds(r, CHUNK)], zbuf)
            pltpu.sync_copy(zbuf, out_hbm.at[pl.ds(r, CHUNK)])
            return carry
        lax.fori_loop(0, NROWS_T // CHUNK, obody, 0)

    @pl.when(c == 0)
    def _():
        write_out(outa_ref)

    @pl.when(c == 1)
    def _():
        write_out(outb_ref)


_deg_call = pl.kernel(
    _deg_body,
    out_type=(jax.ShapeDtypeStruct((NP, CHUNK), F32),
              jax.ShapeDtypeStruct((NP, CHUNK), F32)),
    mesh=_mesh,
    scratch_types=[
        pltpu.VMEM((CHUNK, CHUNK), F32),
        pltpu.VMEM((CHUNK, CHUNK), F32),
        pltpu.VMEM((1, CHUNK), I32),
        pltpu.VMEM_SHARED((NP, CHUNK), F32),
    ],
)


def _rows(bshape):
    return pl.BlockSpec(bshape, lambda i: (i,) + (0,) * (len(bshape) - 1))


def _full(shape):
    return pl.BlockSpec(shape, lambda i: (0,) * len(shape))


def _linear_body(x_ref, w_ref, b_ref, o_ref):
    o_ref[...] = (jnp.dot(x_ref[...], w_ref[...], preferred_element_type=F32)
                  + b_ref[...])


def _linear(x, w, b):
    n, cin = x.shape
    cout = w.shape[1]
    return pl.pallas_call(
        _linear_body,
        grid=(n // RB,),
        in_specs=[_rows((RB, cin)), _full((cin, cout)), _full((1, cout))],
        out_specs=_rows((RB, cout)),
        out_shape=jax.ShapeDtypeStruct((n, cout), F32),
    )(x, w, b.reshape(1, cout))


def _dinv_body(dega_ref, degb_ref, o_ref):
    o_ref[...] = lax.rsqrt(
        dega_ref[...][:, :16] + degb_ref[...][:, :16] + 1.0)


def _dinv(dega, degb):
    return pl.pallas_call(
        _dinv_body,
        grid=(NP // RB,),
        in_specs=[_rows((RB, CHUNK)), _rows((RB, CHUNK))],
        out_specs=_rows((RB, 16)),
        out_shape=jax.ShapeDtypeStruct((NP, 16), F32),
    )(dega, degb)


def _tbias_body(t_ref, w1_ref, b1_ref, w2_ref, b2_ref, wc_ref, bc_ref, o_ref):
    a = t_ref[0, 0] * w1_ref[...] + b1_ref[...]
    a = a * jax.nn.sigmoid(a)
    te = jnp.dot(a, w2_ref[...], preferred_element_type=F32) + b2_ref[...]
    o_ref[...] = jnp.dot(te, wc_ref[...], preferred_element_type=F32) + bc_ref[...]


def _tbias(t2, w1, b1, w2, b2, wc, bc):
    return pl.pallas_call(
        _tbias_body,
        grid=(1,),
        in_specs=[_full((1, 1)), _full((1, H)), _full((1, H)),
                  _full((H, H)), _full((1, H)),
                  _full((H, NCONV * H)), _full((1, NCONV * H))],
        out_specs=_full((1, NCONV * H)),
        out_shape=jax.ShapeDtypeStruct((1, NCONV * H), F32),
    )(t2, w1, b1.reshape(1, H), w2, b2.reshape(1, H), wc, bc)


def _pre256_body(x_ref, g_ref, be_ref, w_ref, dinv_ref, y0_ref, y1_ref):
    x = x_ref[...]
    m = jnp.mean(x, axis=1, keepdims=True)
    v = jnp.mean((x - m) ** 2, axis=1, keepdims=True)
    h = (x - m) * lax.rsqrt(v + 1e-5) * g_ref[...] + be_ref[...]
    y = (jnp.dot(h, w_ref[...], preferred_element_type=F32)
         * dinv_ref[...][:, :1])
    y0_ref[...] = y[:, :HH]
    y1_ref[...] = y[:, HH:]


def _pre256(x, g, be, w, dinv16):
    return pl.pallas_call(
        _pre256_body,
        grid=(NP // RB,),
        in_specs=[_rows((RB, H)), _full((1, H)), _full((1, H)),
                  _full((H, H)), _rows((RB, 16))],
        out_specs=(_rows((RB, HH)), _rows((RB, HH))),
        out_shape=(jax.ShapeDtypeStruct((NP, HH), F32),
                   jax.ShapeDtypeStruct((NP, HH), F32)),
    )(x, g.reshape(1, H), be.reshape(1, H), w, dinv16)


def _pre512_body(x_ref, s_ref, g_ref, be_ref, wx_ref, ws_ref,
                 kx_ref, ks_ref, kb_ref, dinv_ref,
                 y0_ref, y1_ref, sk_ref):
    x = x_ref[...]
    sk = s_ref[...]
    m = (jnp.sum(x, axis=1, keepdims=True)
         + jnp.sum(sk, axis=1, keepdims=True)) / (2.0 * H)
    v = (jnp.sum((x - m) ** 2, axis=1, keepdims=True)
         + jnp.sum((sk - m) ** 2, axis=1, keepdims=True)) / (2.0 * H)
    rs = lax.rsqrt(v + 1e-5)
    g = g_ref[...]
    be = be_ref[...]
    hx = (x - m) * rs * g[:, :H] + be[:, :H]
    hs = (sk - m) * rs * g[:, H:] + be[:, H:]
    y = ((jnp.dot(hx, wx_ref[...], preferred_element_type=F32)
          + jnp.dot(hs, ws_ref[...], preferred_element_type=F32))
         * dinv_ref[...][:, :1])
    y0_ref[...] = y[:, :HH]
    y1_ref[...] = y[:, HH:]
    sk_ref[...] = (jnp.dot(x, kx_ref[...], preferred_element_type=F32)
                   + jnp.dot(sk, ks_ref[...], preferred_element_type=F32)
                   + kb_ref[...])


def _pre512(x, s, g, be, w, kw, kb, dinv16):
    return pl.pallas_call(
        _pre512_body,
        grid=(NP // RB,),
        in_specs=[_rows((RB, H)), _rows((RB, H)),
                  _full((1, 2 * H)), _full((1, 2 * H)),
                  _full((H, H)), _full((H, H)),
                  _full((H, H)), _full((H, H)), _full((1, H)),
                  _rows((RB, 16))],
        out_specs=(_rows((RB, HH)), _rows((RB, HH)), _rows((RB, H))),
        out_shape=(jax.ShapeDtypeStruct((NP, HH), F32),
                   jax.ShapeDtypeStruct((NP, HH), F32),
                   jax.ShapeDtypeStruct((NP, H), F32)),
    )(x, s, g.reshape(1, 2 * H), be.reshape(1, 2 * H),
      w[:H], w[H:], kw[:H], kw[H:], kb.reshape(1, H), dinv16)


def _post_body(u0_ref, u1_ref, dinv_ref, b_ref, tb_ref, o_ref):
    u = jnp.concatenate([u0_ref[...], u1_ref[...]], axis=1)
    a = u * dinv_ref[...][:, :1] + b_ref[...] + tb_ref[...]
    o_ref[...] = a * jax.nn.sigmoid(a)


def _post(u0, u1, dinv16, b, tb):
    return pl.pallas_call(
        _post_body,
        grid=(NP // RB,),
        in_specs=[_rows((RB, HH)), _rows((RB, HH)), _rows((RB, 16)),
                  _full((1, H)), _full((1, H))],
        out_specs=_rows((RB, H)),
        out_shape=jax.ShapeDtypeStruct((NP, H), F32),
    )(u0, u1, dinv16, b.reshape(1, H), tb)


def _post_skip_body(u0_ref, u1_ref, dinv_ref, b_ref, tb_ref, skip_ref, o_ref):
    u = jnp.concatenate([u0_ref[...], u1_ref[...]], axis=1)
    a = u * dinv_ref[...][:, :1] + b_ref[...] + tb_ref[...]
    o_ref[...] = a * jax.nn.sigmoid(a) + skip_ref[...]


def _post_skip(u0, u1, dinv16, b, tb, skip):
    return pl.pallas_call(
        _post_skip_body,
        grid=(NP // RB,),
        in_specs=[_rows((RB, HH)), _rows((RB, HH)), _rows((RB, 16)),
                  _full((1, H)), _full((1, H)), _rows((RB, H))],
        out_specs=_rows((RB, H)),
        out_shape=jax.ShapeDtypeStruct((NP, H), F32),
    )(u0, u1, dinv16, b.reshape(1, H), tb, skip)


def kernel(x, t, edge_index, params):
    src = edge_index[0].astype(I32)
    dst = edge_index[1].astype(I32)
    pad_e = EP - E
    srcp = jnp.concatenate(
        [src, jnp.zeros((pad_e,), I32)]).reshape(EP // CHUNK, CHUNK)
    dstp = jnp.concatenate(
        [dst, jnp.full((pad_e,), NP - 1, I32)]).reshape(EP // CHUNK, CHUNK)
    xp = jnp.pad(x.astype(F32), ((0, NP - N), (0, 0)))
    P = params

    ones = jnp.ones((CHUNK, CHUNK), F32)
    zeros = jnp.zeros((CHUNK, CHUNK), F32)
    dega, degb = _deg_call(dstp, ones, zeros)
    dinv16 = _dinv(dega, degb)

    blocks = list(P['down']) + [P['mid']] + list(P['up'])
    twc = jnp.concatenate(
        [w for p in blocks for w in (p['tW1'], p['tW2'])], axis=1)
    tbc = jnp.concatenate(
        [b.reshape(1, H) for p in blocks for b in (p['tb1'], p['tb2'])],
        axis=1)
    t2 = jnp.reshape(t, (1, 1)).astype(F32)
    tbias = _tbias(t2, P['time_W1'], P['time_b1'], P['time_W2'],
                   P['time_b2'], twc, tbc)

    h = _linear(xp, P['in_W'], P['in_b'])

    ci = [0]

    def run_block(h, s, p):
        if s is None:
            y0, y1 = _pre256(h, p['g1'], p['be1'], p['W1'], dinv16)
            skip = h
        else:
            y0, y1, skip = _pre512(h, s, p['g1'], p['be1'], p['W1'],
                                   p['skipW'], p['skipb'], dinv16)
        u0, u1 = _spmm_call(srcp, dstp, y0, y1)
        tb1 = tbias[:, ci[0] * H:(ci[0] + 1) * H]
        ci[0] += 1
        z = _post(u0, u1, dinv16, p['b1'], tb1)
        y0, y1 = _pre256(z, p['g2'], p['be2'], p['W2'], dinv16)
        u0, u1 = _spmm_call(srcp, dstp, y0, y1)
        tb2 = tbias[:, ci[0] * H:(ci[0] + 1) * H]
        ci[0] += 1
        return _post_skip(u0, u1, dinv16, p['b2'], tb2, skip)

    skips = [h]
    for p in P['down']:
        h = run_block(h, None, p)
        skips.append(h)
    h = run_block(h, None, P['mid'])
    for p, s in zip(P['up'], reversed(skips)):
        h = run_block(h, s, p)

    out = _linear(h, P['out_W'], P['out_b'])
    return out[:N]


# speedup vs baseline: 1.2816x; 1.2816x over previous
import jax
import jax.numpy as jnp
from jax import lax
from jax.experimental import pallas as pl
from jax.experimental.pallas import tpu as pltpu
from jax.experimental.pallas import tpu_sc as plsc

F32 = jnp.float32
I32 = jnp.int32

N = 10000
E = 320000
D = 128
H = 256
HH = H // 2
NP = 10240
CHUNK = 128
CPT = 157
EP = 16 * CHUNK * CPT
NROWS_T = NP // 16
RB = 256
NCONV = 18

_mesh = plsc.VectorSubcoreMesh(core_axis_name="c", subcore_axis_name="s")


def _spmm_body(src_ref, dst_ref, y0_ref, y1_ref, out0_ref, out1_ref,
               stage, sidx, didx, acc, sem):
    c = lax.axis_index("c")
    s = lax.axis_index("s")
    r0 = s * NROWS_T

    def run(y_hbm, out_hbm):
        def init_body(j, carry):
            r = r0 + j * CHUNK
            pltpu.sync_copy(y_hbm.at[pl.ds(r, CHUNK)], stage)
            pltpu.sync_copy(stage, acc.at[pl.ds(r, CHUNK)])
            return carry
        lax.fori_loop(0, NROWS_T // CHUNK, init_body, 0)
        plsc.subcore_barrier()

        base = s * CPT

        def edge_body(i, carry):
            row = base + i
            pltpu.sync_copy(src_ref.at[row], sidx)
            pltpu.sync_copy(dst_ref.at[pl.ds(row, 1)], didx)
            pltpu.async_copy(y_hbm.at[sidx], stage, sem).wait()
            pltpu.sync_copy(stage, acc.at[didx.at[0]], add=True)
            return carry
        lax.fori_loop(0, CPT, edge_body, 0)
        plsc.subcore_barrier()

        def out_body(j, carry):
            r = r0 + j * CHUNK
            pltpu.sync_copy(acc.at[pl.ds(r, CHUNK)], stage)
            pltpu.sync_copy(stage, out_hbm.at[pl.ds(r, CHUNK)])
            return carry
        lax.fori_loop(0, NROWS_T // CHUNK, out_body, 0)

    @pl.when(c == 0)
    def _():
        run(y0_ref, out0_ref)

    @pl.when(c == 1)
    def _():
        run(y1_ref, out1_ref)


_spmm_call = pl.kernel(
    _spmm_body,
    out_type=(jax.ShapeDtypeStruct((NP, HH), F32),
              jax.ShapeDtypeStruct((NP, HH), F32)),
    mesh=_mesh,
    scratch_types=[
        pltpu.VMEM((CHUNK, HH), F32),
        pltpu.VMEM((CHUNK,), I32),
        pltpu.VMEM((1, CHUNK), I32),
        pltpu.VMEM_SHARED((NP, HH), F32),
        pltpu.SemaphoreType.DMA,
    ],
)


def _deg_body(dst_ref, ones_ref, zeros_ref, out_ref, obuf, zbuf, didx, acc):
    c = lax.axis_index("c")
    s = lax.axis_index("s")
    r0 = s * NROWS_T

    @pl.when(c == 0)
    def _():
        pltpu.sync_copy(ones_ref, obuf)
        pltpu.sync_copy(zeros_ref, zbuf)

        def zbody(j, carry):
            r = r0 + j * CHUNK
            pltpu.sync_copy(zbuf, acc.at[pl.ds(r, CHUNK)])
            return carry
        lax.fori_loop(0, NROWS_T // CHUNK, zbody, 0)
        plsc.subcore_barrier()

        base = s * CPT

        def ebody(i, carry):
            row = base + i
            pltpu.sync_copy(dst_ref.at[pl.ds(row, 1)], didx)
            pltpu.sync_copy(obuf, acc.at[didx.at[0]], add=True)
            return carry
        lax.fori_loop(0, CPT, ebody, 0)
        plsc.subcore_barrier()

        def obody(j, carry):
            r = r0 + j * CHUNK
            pltpu.sync_copy(acc.at[pl.ds(r, CHUNK)], zbuf)
            pltpu.sync_copy(zbuf, out_ref.at[pl.ds(r, CHUNK)])
            return carry
        lax.fori_loop(0, NROWS_T // CHUNK, obody, 0)


_deg_call = pl.kernel(
    _deg_body,
    out_type=jax.ShapeDtypeStruct((NP, CHUNK), F32),
    mesh=_mesh,
    scratch_types=[
        pltpu.VMEM((CHUNK, CHUNK), F32),
        pltpu.VMEM((CHUNK, CHUNK), F32),
        pltpu.VMEM((1, CHUNK), I32),
        pltpu.VMEM_SHARED((NP, CHUNK), F32),
    ],
)


def _rows(bshape):
    return pl.BlockSpec(bshape, lambda i: (i,) + (0,) * (len(bshape) - 1))


def _full(shape):
    return pl.BlockSpec(shape, lambda i: (0,) * len(shape))


def _linear_body(x_ref, w_ref, b_ref, o_ref):
    o_ref[...] = (jnp.dot(x_ref[...], w_ref[...], preferred_element_type=F32)
                  + b_ref[...])


def _linear(x, w, b):
    n, cin = x.shape
    cout = w.shape[1]
    return pl.pallas_call(
        _linear_body,
        grid=(n // RB,),
        in_specs=[_rows((RB, cin)), _full((cin, cout)), _full((1, cout))],
        out_specs=_rows((RB, cout)),
        out_shape=jax.ShapeDtypeStruct((n, cout), F32),
    )(x, w, b.reshape(1, cout))


def _dinv_body(deg_ref, o_ref):
    o_ref[...] = lax.rsqrt(deg_ref[...][:, :16] + 1.0)


def _dinv(deg128):
    return pl.pallas_call(
        _dinv_body,
        grid=(NP // RB,),
        in_specs=[_rows((RB, CHUNK))],
        out_specs=_rows((RB, 16)),
        out_shape=jax.ShapeDtypeStruct((NP, 16), F32),
    )(deg128)


def _tbias_body(t_ref, w1_ref, b1_ref, w2_ref, b2_ref, wc_ref, bc_ref, o_ref):
    a = t_ref[0, 0] * w1_ref[...] + b1_ref[...]
    a = a * jax.nn.sigmoid(a)
    te = jnp.dot(a, w2_ref[...], preferred_element_type=F32) + b2_ref[...]
    o_ref[...] = jnp.dot(te, wc_ref[...], preferred_element_type=F32) + bc_ref[...]


def _tbias(t2, w1, b1, w2, b2, wc, bc):
    return pl.pallas_call(
        _tbias_body,
        grid=(1,),
        in_specs=[_full((1, 1)), _full((1, H)), _full((1, H)),
                  _full((H, H)), _full((1, H)),
                  _full((H, NCONV * H)), _full((1, NCONV * H))],
        out_specs=_full((1, NCONV * H)),
        out_shape=jax.ShapeDtypeStruct((1, NCONV * H), F32),
    )(t2, w1, b1.reshape(1, H), w2, b2.reshape(1, H), wc, bc)


def _pre256_body(x_ref, g_ref, be_ref, w_ref, dinv_ref, y0_ref, y1_ref):
    x = x_ref[...]
    m = jnp.mean(x, axis=1, keepdims=True)
    v = jnp.mean((x - m) ** 2, axis=1, keepdims=True)
    h = (x - m) * lax.rsqrt(v + 1e-5) * g_ref[...] + be_ref[...]
    y = (jnp.dot(h, w_ref[...], preferred_element_type=F32)
         * dinv_ref[...][:, :1])
    y0_ref[...] = y[:, :HH]
    y1_ref[...] = y[:, HH:]


def _pre256(x, g, be, w, dinv16):
    return pl.pallas_call(
        _pre256_body,
        grid=(NP // RB,),
        in_specs=[_rows((RB, H)), _full((1, H)), _full((1, H)),
                  _full((H, H)), _rows((RB, 16))],
        out_specs=(_rows((RB, HH)), _rows((RB, HH))),
        out_shape=(jax.ShapeDtypeStruct((NP, HH), F32),
                   jax.ShapeDtypeStruct((NP, HH), F32)),
    )(x, g.reshape(1, H), be.reshape(1, H), w, dinv16)


def _pre512_body(x_ref, s_ref, g_ref, be_ref, wx_ref, ws_ref,
                 kx_ref, ks_ref, kb_ref, dinv_ref,
                 y0_ref, y1_ref, sk_ref):
    x = x_ref[...]
    sk = s_ref[...]
    m = (jnp.sum(x, axis=1, keepdims=True)
         + jnp.sum(sk, axis=1, keepdims=True)) / (2.0 * H)
    v = (jnp.sum((x - m) ** 2, axis=1, keepdims=True)
         + jnp.sum((sk - m) ** 2, axis=1, keepdims=True)) / (2.0 * H)
    rs = lax.rsqrt(v + 1e-5)
    g = g_ref[...]
    be = be_ref[...]
    hx = (x - m) * rs * g[:, :H] + be[:, :H]
    hs = (sk - m) * rs * g[:, H:] + be[:, H:]
    y = ((jnp.dot(hx, wx_ref[...], preferred_element_type=F32)
          + jnp.dot(hs, ws_ref[...], preferred_element_type=F32))
         * dinv_ref[...][:, :1])
    y0_ref[...] = y[:, :HH]
    y1_ref[...] = y[:, HH:]
    sk_ref[...] = (jnp.dot(x, kx_ref[...], preferred_element_type=F32)
                   + jnp.dot(sk, ks_ref[...], preferred_element_type=F32)
                   + kb_ref[...])


def _pre512(x, s, g, be, w, kw, kb, dinv16):
    return pl.pallas_call(
        _pre512_body,
        grid=(NP // RB,),
        in_specs=[_rows((RB, H)), _rows((RB, H)),
                  _full((1, 2 * H)), _full((1, 2 * H)),
                  _full((H, H)), _full((H, H)),
                  _full((H, H)), _full((H, H)), _full((1, H)),
                  _rows((RB, 16))],
        out_specs=(_rows((RB, HH)), _rows((RB, HH)), _rows((RB, H))),
        out_shape=(jax.ShapeDtypeStruct((NP, HH), F32),
                   jax.ShapeDtypeStruct((NP, HH), F32),
                   jax.ShapeDtypeStruct((NP, H), F32)),
    )(x, s, g.reshape(1, 2 * H), be.reshape(1, 2 * H),
      w[:H], w[H:], kw[:H], kw[H:], kb.reshape(1, H), dinv16)


def _post_body(u0_ref, u1_ref, dinv_ref, b_ref, tb_ref, o_ref):
    u = jnp.concatenate([u0_ref[...], u1_ref[...]], axis=1)
    a = u * dinv_ref[...][:, :1] + b_ref[...] + tb_ref[...]
    o_ref[...] = a * jax.nn.sigmoid(a)


def _post(u0, u1, dinv16, b, tb):
    return pl.pallas_call(
        _post_body,
        grid=(NP // RB,),
        in_specs=[_rows((RB, HH)), _rows((RB, HH)), _rows((RB, 16)),
                  _full((1, H)), _full((1, H))],
        out_specs=_rows((RB, H)),
        out_shape=jax.ShapeDtypeStruct((NP, H), F32),
    )(u0, u1, dinv16, b.reshape(1, H), tb)


def _post_skip_body(u0_ref, u1_ref, dinv_ref, b_ref, tb_ref, skip_ref, o_ref):
    u = jnp.concatenate([u0_ref[...], u1_ref[...]], axis=1)
    a = u * dinv_ref[...][:, :1] + b_ref[...] + tb_ref[...]
    o_ref[...] = a * jax.nn.sigmoid(a) + skip_ref[...]


def _post_skip(u0, u1, dinv16, b, tb, skip):
    return pl.pallas_call(
        _post_skip_body,
        grid=(NP // RB,),
        in_specs=[_rows((RB, HH)), _rows((RB, HH)), _rows((RB, 16)),
                  _full((1, H)), _full((1, H)), _rows((RB, H))],
        out_specs=_rows((RB, H)),
        out_shape=jax.ShapeDtypeStruct((NP, H), F32),
    )(u0, u1, dinv16, b.reshape(1, H), tb, skip)


def kernel(x, t, edge_index, params):
    src = edge_index[0].astype(I32)
    dst = edge_index[1].astype(I32)
    pad_e = EP - E
    srcp = jnp.concatenate(
        [src, jnp.zeros((pad_e,), I32)]).reshape(EP // CHUNK, CHUNK)
    dstp = jnp.concatenate(
        [dst, jnp.full((pad_e,), NP - 1, I32)]).reshape(EP // CHUNK, CHUNK)
    xp = jnp.pad(x.astype(F32), ((0, NP - N), (0, 0)))
    P = params

    ones = jnp.ones((CHUNK, CHUNK), F32)
    zeros = jnp.zeros((CHUNK, CHUNK), F32)
    deg128 = _deg_call(dstp, ones, zeros)
    dinv16 = _dinv(deg128)

    blocks = list(P['down']) + [P['mid']] + list(P['up'])
    twc = jnp.concatenate(
        [w for p in blocks for w in (p['tW1'], p['tW2'])], axis=1)
    tbc = jnp.concatenate(
        [b.reshape(1, H) for p in blocks for b in (p['tb1'], p['tb2'])],
        axis=1)
    t2 = jnp.reshape(t, (1, 1)).astype(F32)
    tbias = _tbias(t2, P['time_W1'], P['time_b1'], P['time_W2'],
                   P['time_b2'], twc, tbc)

    h = _linear(xp, P['in_W'], P['in_b'])

    ci = [0]

    def run_block(h, s, p):
        if s is None:
            y0, y1 = _pre256(h, p['g1'], p['be1'], p['W1'], dinv16)
            skip = h
        else:
            y0, y1, skip = _pre512(h, s, p['g1'], p['be1'], p['W1'],
                                   p['skipW'], p['skipb'], dinv16)
        u0, u1 = _spmm_call(srcp, dstp, y0, y1)
        tb1 = tbias[:, ci[0] * H:(ci[0] + 1) * H]
        ci[0] += 1
        z = _post(u0, u1, dinv16, p['b1'], tb1)
        y0, y1 = _pre256(z, p['g2'], p['be2'], p['W2'], dinv16)
        u0, u1 = _spmm_call(srcp, dstp, y0, y1)
        tb2 = tbias[:, ci[0] * H:(ci[0] + 1) * H]
        ci[0] += 1
        return _post_skip(u0, u1, dinv16, p['b2'], tb2, skip)

    skips = [h]
    for p in P['down']:
        h = run_block(h, None, p)
        skips.append(h)
    h = run_block(h, None, P['mid'])
    for p, s in zip(P['up'], reversed(skips)):
        h = run_block(h, s, p)

    out = _linear(h, P['out_W'], P['out_b'])
    return out[:N]
